# baseline (speedup 1.0000x reference)
import functools

import jax
import jax.numpy as jnp
from jax import lax
from jax.experimental import pallas as pl
from jax.experimental.pallas import tpu as pltpu
from jax.experimental.pallas import tpu_sc as plsc

NC = 2
NS = 16
NW = NC * NS
B = 128
H = 128

F32 = jnp.float32


def _mesh():
    return plsc.VectorSubcoreMesh(core_axis_name="c", subcore_axis_name="s")


def _zero_vmem(ref, rows, cols):
    z = jnp.zeros((16,), F32)
    cchunks = cols // 16

    def body(i, carry):
        ref[i // cchunks, pl.ds((i % cchunks) * 16, 16)] = z
        return carry

    lax.fori_loop(0, rows * cchunks, body, 0)


def _fill_vmem(ref, rows, cols, val):
    v = jnp.full((16,), val, F32)
    cchunks = cols // 16

    def body(i, carry):
        ref[i // cchunks, pl.ds((i % cchunks) * 16, 16)] = v
        return carry

    lax.fori_loop(0, rows * cchunks, body, 0)


def _stripe_chunks(stripe):
    chunks = []
    off = 0
    while off < stripe:
        sz = min(B, stripe - off)
        chunks.append((off, sz))
        off += sz
    return chunks


def _make_segsum_kernel(Kn, Ks, src_rows, dst_rows):
    stripe = dst_rows // NS
    chunks = _stripe_chunks(stripe)
    out_t = jax.ShapeDtypeStruct((NC, dst_rows, H), F32)
    K = max(Kn, Ks)

    assert Kn % 2 == 0 and Ks % 2 == 0

    @functools.partial(
        pl.kernel,
        out_type=(out_t, out_t),
        mesh=_mesh(),
        scratch_types=[
            pltpu.VMEM((K, B), jnp.int32),
            pltpu.VMEM((B,), jnp.int32),
            pltpu.VMEM((B,), jnp.int32),
            pltpu.VMEM((B, H), F32),
            pltpu.VMEM((B, H), F32),
            pltpu.SemaphoreType.DMA,
            pltpu.SemaphoreType.DMA,
            pltpu.SemaphoreType.DMA,
            pltpu.SemaphoreType.DMA,
            pltpu.VMEM_SHARED((dst_rows, H), F32),
        ],
    )
    def segsum_kernel(s0, s1, gi_hbm, si_hbm, o0, o1,
                      gi_v, sb0, sb1, buf0, buf1, sg0, sg1, si0, si1, acc_sh):
        cid = lax.axis_index("c")
        sid = lax.axis_index("s")
        wid = cid * NS + sid
        kw = jnp.where(cid == 0, Kn, Ks)

        pltpu.sync_copy(gi_hbm.at[wid], gi_v)

        for src_hbm, out_hbm in ((s0, o0), (s1, o1)):
            _zero_vmem(buf0, B, H)
            for off, sz in chunks:
                pltpu.sync_copy(buf0.at[pl.ds(0, sz)],
                                acc_sh.at[pl.ds(sid * stripe + off, sz)])
            plsc.subcore_barrier()

            def start(j, sb, buf, sg, si_sem):
                gdesc = pltpu.async_copy(src_hbm.at[gi_v.at[j]], buf, sg)
                idesc = pltpu.async_copy(si_hbm.at[wid].at[j], sb, si_sem)
                return gdesc, idesc

            def finish(sb, buf, gdesc, idesc):
                gdesc.wait()
                idesc.wait()
                pltpu.sync_copy(buf, acc_sh.at[sb], add=True)

            g0, i0 = start(0, sb0, buf0, sg0, si0)

            def body(i, carry):
                j0 = 2 * i
                g0d, i0d = pltpu.make_async_copy(src_hbm.at[gi_v.at[j0]], buf0, sg0), \
                    pltpu.make_async_copy(si_hbm.at[wid].at[j0], sb0, si0)
                g1d = pltpu.async_copy(src_hbm.at[gi_v.at[j0 + 1]], buf1, sg1)
                i1d = pltpu.async_copy(si_hbm.at[wid].at[j0 + 1], sb1, si1)
                finish(sb0, buf0, g0d, i0d)

                @pl.when(j0 + 2 < kw)
                def _():
                    pltpu.async_copy(src_hbm.at[gi_v.at[j0 + 2]], buf0, sg0)
                    pltpu.async_copy(si_hbm.at[wid].at[j0 + 2], sb0, si0)

                finish(sb1, buf1, g1d, i1d)
                return carry

            lax.fori_loop(0, kw // 2, body, 0)
            plsc.subcore_barrier()

            for off, sz in chunks:
                base = sid * stripe + off
                pltpu.sync_copy(acc_sh.at[pl.ds(base, sz)],
                                buf0.at[pl.ds(0, sz)])
                pltpu.sync_copy(buf0.at[pl.ds(0, sz)],
                                out_hbm.at[cid].at[pl.ds(base, sz)])

    return segsum_kernel


def _make_degree_kernel(K, NP, NEP):
    dv_stripe = NP // NS
    de_stripe = NEP // NS
    dv_chunks = _stripe_chunks(dv_stripe)
    de_chunks = _stripe_chunks(de_stripe)

    @functools.partial(
        pl.kernel,
        out_type=(
            jax.ShapeDtypeStruct((NC, NP, H), F32),
            jax.ShapeDtypeStruct((NC, NEP, H), F32),
        ),
        mesh=_mesh(),
        scratch_types=[
            pltpu.VMEM((B,), jnp.int32),
            pltpu.VMEM((B,), jnp.int32),
            pltpu.VMEM((B, H), F32),
            pltpu.VMEM_SHARED((NP, H), F32),
            pltpu.VMEM_SHARED((NEP, H), F32),
        ],
    )
    def deg_kernel(ni_hbm, hi_hbm, dv_out, de_out,
                   nb_v, hb_v, ones_v, dv_sh, de_sh):
        cid = lax.axis_index("c")
        sid = lax.axis_index("s")
        wid = cid * NS + sid

        _zero_vmem(ones_v, B, H)
        for off, sz in dv_chunks:
            pltpu.sync_copy(ones_v.at[pl.ds(0, sz)],
                            dv_sh.at[pl.ds(sid * dv_stripe + off, sz)])
        for off, sz in de_chunks:
            pltpu.sync_copy(ones_v.at[pl.ds(0, sz)],
                            de_sh.at[pl.ds(sid * de_stripe + off, sz)])
        _fill_vmem(ones_v, B, H, 1.0)
        plsc.subcore_barrier()

        def body(j, carry):
            pltpu.sync_copy(ni_hbm.at[wid].at[j], nb_v)
            pltpu.sync_copy(hi_hbm.at[wid].at[j], hb_v)
            pltpu.sync_copy(ones_v, dv_sh.at[nb_v], add=True)
            pltpu.sync_copy(ones_v, de_sh.at[hb_v], add=True)
            return carry

        lax.fori_loop(0, K, body, 0)
        plsc.subcore_barrier()

        for off, sz in dv_chunks:
            base = sid * dv_stripe + off
            pltpu.sync_copy(dv_sh.at[pl.ds(base, sz)], ones_v.at[pl.ds(0, sz)])
            pltpu.sync_copy(ones_v.at[pl.ds(0, sz)],
                            dv_out.at[cid].at[pl.ds(base, sz)])
        for off, sz in de_chunks:
            base = sid * de_stripe + off
            pltpu.sync_copy(de_sh.at[pl.ds(base, sz)], ones_v.at[pl.ds(0, sz)])
            pltpu.sync_copy(ones_v.at[pl.ds(0, sz)],
                            de_out.at[cid].at[pl.ds(base, sz)])

    return deg_kernel


def _scale_from_parts(parts, power):
    d = parts[0] + parts[1]
    if power == -1.0:
        s = 1.0 / d
    else:
        s = lax.rsqrt(d)
    s = jnp.where(d > 0, s, 0.0)
    return s[:, 0:1]


def _mlp1_body(x_ref, w_ref, b_ref, dvp_ref, y0, y1):
    s = _scale_from_parts(dvp_ref[...], -0.5)
    y = (jnp.dot(x_ref[...], w_ref[...], preferred_element_type=F32)
         + b_ref[...]) * s
    y0[...] = y[:, :H]
    y1[...] = y[:, H:]


def _escale_body(e0p, e1p, dep_ref, e0, e1):
    s = _scale_from_parts(dep_ref[...], -1.0)
    e0[...] = (e0p[0] + e0p[1]) * s
    e1[...] = (e1p[0] + e1p[1]) * s


def _mlp2_body(z0p, z1p, dvp_ref, w2_ref, b_ref, y0, y1):
    s = _scale_from_parts(dvp_ref[...], -0.5)
    h0 = jnp.maximum((z0p[0] + z0p[1]) * s, 0.0)
    h1 = jnp.maximum((z1p[0] + z1p[1]) * s, 0.0)
    y = (jnp.dot(h0, w2_ref[:H, :], preferred_element_type=F32)
         + jnp.dot(h1, w2_ref[H:, :], preferred_element_type=F32)
         + b_ref[...]) * s
    y0[...] = y[:, :H]
    y1[...] = y[:, H:]


def _final_body(z0p, z1p, dvp_ref, o0, o1):
    s = _scale_from_parts(dvp_ref[...], -0.5)
    o0[...] = (z0p[0] + z0p[1]) * s
    o1[...] = (z1p[0] + z1p[1]) * s


def kernel(X, node_idx, hedge_idx, W1, b1, W2, b2):
    N, D = X.shape
    HID = W1.shape[1]
    OUT = W2.shape[1]
    NNZ = node_idx.shape[0]
    NE = 2500

    def rup(x, m):
        return (x + m - 1) // m * m

    NP = rup(N + 1, NS * B)
    NEP = rup(NE + 1, B)
    K = (NNZ + NW * B - 1) // (NW * B)
    tot = NW * K * B

    npad = tot - NNZ
    ni_pad = (N + jnp.arange(npad, dtype=jnp.int32) % (NP - N))
    hi_pad = (NE + jnp.arange(npad, dtype=jnp.int32) % (NEP - NE))
    ni_flat = jnp.concatenate([node_idx.astype(jnp.int32), ni_pad])
    hi_flat = jnp.concatenate([hedge_idx.astype(jnp.int32), hi_pad])
    ni = ni_flat.reshape(NW, K, B)
    hi = hi_flat.reshape(NW, K, B)

    nb = tot // B
    Kn = rup(int(round(nb * 0.45)) // NS, 2)
    Ks = rup((nb - Kn * NS + NS - 1) // NS, 2)
    Kmax = max(Kn, Ks)

    def asym(flat):
        blocks = flat.reshape(nb, B)
        north = blocks[:Kn * NS].reshape(NS, Kn, B)
        south = blocks[Kn * NS:Kn * NS + Ks * NS]
        south = jnp.concatenate(
            [south, jnp.tile(blocks[-1:], (Ks * NS - (nb - Kn * NS), 1))]
        )[: Ks * NS].reshape(NS, Ks, B) if (nb - Kn * NS) < Ks * NS else \
            south.reshape(NS, Ks, B)
        if Ks < Kmax:
            south = jnp.concatenate(
                [south, jnp.tile(south[:, -1:], (1, Kmax - Ks, 1))], axis=1)
        if Kn < Kmax:
            north = jnp.concatenate(
                [north, jnp.tile(north[:, -1:], (1, Kmax - Kn, 1))], axis=1)
        return jnp.concatenate([north, south], axis=0)

    Xp = jnp.pad(X, ((0, NP - N), (0, 0)))
    b1r = b1.reshape(1, HID)
    b2r = b2.reshape(1, OUT)

    ni_a = asym(ni_flat)
    hi_a = asym(hi_flat)

    dv_parts, de_parts = _make_degree_kernel(K, NP, NEP)(ni, hi)

    RB = 1024
    G = NP // RB
    h_out = lambda R: [jax.ShapeDtypeStruct((R, H), F32)] * 2

    mlp1 = pl.pallas_call(
        _mlp1_body,
        grid=(G,),
        in_specs=[
            pl.BlockSpec((RB, D), lambda i: (i, 0)),
            pl.BlockSpec((D, HID), lambda i: (0, 0)),
            pl.BlockSpec((1, HID), lambda i: (0, 0)),
            pl.BlockSpec((NC, RB, H), lambda i: (0, i, 0)),
        ],
        out_specs=[pl.BlockSpec((RB, H), lambda i: (i, 0))] * 2,
        out_shape=h_out(NP),
    )

    escale = pl.pallas_call(
        _escale_body,
        grid=(1,),
        in_specs=[pl.BlockSpec((NC, NEP, H), lambda i: (0, 0, 0))] * 2
        + [pl.BlockSpec((NC, NEP, H), lambda i: (0, 0, 0))],
        out_specs=[pl.BlockSpec((NEP, H), lambda i: (0, 0))] * 2,
        out_shape=h_out(NEP),
    )

    mlp2 = pl.pallas_call(
        _mlp2_body,
        grid=(G,),
        in_specs=[pl.BlockSpec((NC, RB, H), lambda i: (0, i, 0))] * 2
        + [
            pl.BlockSpec((NC, RB, H), lambda i: (0, i, 0)),
            pl.BlockSpec((HID, OUT), lambda i: (0, 0)),
            pl.BlockSpec((1, OUT), lambda i: (0, 0)),
        ],
        out_specs=[pl.BlockSpec((RB, H), lambda i: (i, 0))] * 2,
        out_shape=h_out(NP),
    )

    final = pl.pallas_call(
        _final_body,
        grid=(G,),
        in_specs=[pl.BlockSpec((NC, RB, H), lambda i: (0, i, 0))] * 2
        + [pl.BlockSpec((NC, RB, H), lambda i: (0, i, 0))],
        out_specs=[pl.BlockSpec((RB, H), lambda i: (i, 0))] * 2,
        out_shape=h_out(NP),
    )

    seg_e = _make_segsum_kernel(Kn, Ks, NP, NEP)
    seg_z = _make_segsum_kernel(Kn, Ks, NEP, NP)

    def laplacian(yh):
        eph = seg_e(*yh, ni_a, hi_a)
        eh = escale(*eph, de_parts)
        return seg_z(*eh, hi_a, ni_a)

    yh = mlp1(Xp, W1, b1r, dv_parts)
    zph = laplacian(yh)
    y2h = mlp2(*zph, dv_parts, W2, b2r)
    zph2 = laplacian(y2h)
    o0, o1 = final(*zph2, dv_parts)
    return jnp.concatenate([o0[:N], o1[:N]], axis=1)


# speedup vs baseline: 1.0499x; 1.0499x over previous
import functools

import jax
import jax.numpy as jnp
from jax import lax
from jax.experimental import pallas as pl
from jax.experimental.pallas import tpu as pltpu
from jax.experimental.pallas import tpu_sc as plsc

NC = 2
NS = 16
NW = NC * NS
B = 128
H = 128

F32 = jnp.float32


def _mesh():
    return plsc.VectorSubcoreMesh(core_axis_name="c", subcore_axis_name="s")


def _zero_vmem(ref, rows, cols):
    z = jnp.zeros((16,), F32)
    cchunks = cols // 16

    def body(i, carry):
        ref[i // cchunks, pl.ds((i % cchunks) * 16, 16)] = z
        return carry

    lax.fori_loop(0, rows * cchunks, body, 0)


def _fill_vmem(ref, rows, cols, val):
    v = jnp.full((16,), val, F32)
    cchunks = cols // 16

    def body(i, carry):
        ref[i // cchunks, pl.ds((i % cchunks) * 16, 16)] = v
        return carry

    lax.fori_loop(0, rows * cchunks, body, 0)


def _stripe_chunks(stripe):
    chunks = []
    off = 0
    while off < stripe:
        sz = min(B, stripe - off)
        chunks.append((off, sz))
        off += sz
    return chunks


def _make_segsum_kernel(Kn, Ks, src_rows, dst_rows):
    stripe = dst_rows // NS
    chunks = _stripe_chunks(stripe)
    out_t = jax.ShapeDtypeStruct((NC, dst_rows, H), F32)
    K = max(Kn, Ks)

    assert Kn % 2 == 0 and Ks % 2 == 0

    @functools.partial(
        pl.kernel,
        out_type=(out_t, out_t),
        mesh=_mesh(),
        scratch_types=[
            pltpu.VMEM((K, B), jnp.int32),
            pltpu.VMEM((B,), jnp.int32),
            pltpu.VMEM((B,), jnp.int32),
            pltpu.VMEM((B, H), F32),
            pltpu.VMEM((B, H), F32),
            pltpu.SemaphoreType.DMA,
            pltpu.SemaphoreType.DMA,
            pltpu.SemaphoreType.DMA,
            pltpu.SemaphoreType.DMA,
            pltpu.VMEM_SHARED((dst_rows, H), F32),
        ],
    )
    def segsum_kernel(s0, s1, gi_hbm, si_hbm, o0, o1,
                      gi_v, sb0, sb1, buf0, buf1, sg0, sg1, si0, si1, acc_sh):
        cid = lax.axis_index("c")
        sid = lax.axis_index("s")
        wid = cid * NS + sid
        kw = jnp.where(cid == 0, Kn, Ks)

        pltpu.sync_copy(gi_hbm.at[wid], gi_v)

        for src_hbm, out_hbm in ((s0, o0), (s1, o1)):
            _zero_vmem(buf0, B, H)
            for off, sz in chunks:
                pltpu.sync_copy(buf0.at[pl.ds(0, sz)],
                                acc_sh.at[pl.ds(sid * stripe + off, sz)])
            plsc.subcore_barrier()

            def start(j, sb, buf, sg, si_sem):
                gdesc = pltpu.async_copy(src_hbm.at[gi_v.at[j]], buf, sg)
                idesc = pltpu.async_copy(si_hbm.at[wid].at[j], sb, si_sem)
                return gdesc, idesc

            def finish(sb, buf, gdesc, idesc):
                gdesc.wait()
                idesc.wait()
                pltpu.sync_copy(buf, acc_sh.at[sb], add=True)

            g0, i0 = start(0, sb0, buf0, sg0, si0)

            def body(i, carry):
                j0 = 2 * i
                g0d, i0d = pltpu.make_async_copy(src_hbm.at[gi_v.at[j0]], buf0, sg0), \
                    pltpu.make_async_copy(si_hbm.at[wid].at[j0], sb0, si0)
                g1d = pltpu.async_copy(src_hbm.at[gi_v.at[j0 + 1]], buf1, sg1)
                i1d = pltpu.async_copy(si_hbm.at[wid].at[j0 + 1], sb1, si1)
                finish(sb0, buf0, g0d, i0d)

                @pl.when(j0 + 2 < kw)
                def _():
                    pltpu.async_copy(src_hbm.at[gi_v.at[j0 + 2]], buf0, sg0)
                    pltpu.async_copy(si_hbm.at[wid].at[j0 + 2], sb0, si0)

                finish(sb1, buf1, g1d, i1d)
                return carry

            lax.fori_loop(0, kw // 2, body, 0)
            plsc.subcore_barrier()

            for off, sz in chunks:
                base = sid * stripe + off
                pltpu.sync_copy(acc_sh.at[pl.ds(base, sz)],
                                buf0.at[pl.ds(0, sz)])
                pltpu.sync_copy(buf0.at[pl.ds(0, sz)],
                                out_hbm.at[cid].at[pl.ds(base, sz)])

    return segsum_kernel


def _make_degree_kernel(K, NP, NEP):
    dv_stripe = NP // NS
    de_stripe = NEP // NS
    dv_chunks = _stripe_chunks(dv_stripe)
    de_chunks = _stripe_chunks(de_stripe)

    @functools.partial(
        pl.kernel,
        out_type=(
            jax.ShapeDtypeStruct((NC, NP, H), F32),
            jax.ShapeDtypeStruct((NC, NEP, H), F32),
        ),
        mesh=_mesh(),
        scratch_types=[
            pltpu.VMEM((B,), jnp.int32),
            pltpu.VMEM((B,), jnp.int32),
            pltpu.VMEM((B, H), F32),
            pltpu.VMEM_SHARED((NP, H), F32),
            pltpu.VMEM_SHARED((NEP, H), F32),
        ],
    )
    def deg_kernel(ni_hbm, hi_hbm, dv_out, de_out,
                   nb_v, hb_v, ones_v, dv_sh, de_sh):
        cid = lax.axis_index("c")
        sid = lax.axis_index("s")
        wid = cid * NS + sid

        _zero_vmem(ones_v, B, H)
        for off, sz in dv_chunks:
            pltpu.sync_copy(ones_v.at[pl.ds(0, sz)],
                            dv_sh.at[pl.ds(sid * dv_stripe + off, sz)])
        for off, sz in de_chunks:
            pltpu.sync_copy(ones_v.at[pl.ds(0, sz)],
                            de_sh.at[pl.ds(sid * de_stripe + off, sz)])
        _fill_vmem(ones_v, B, H, 1.0)
        plsc.subcore_barrier()

        def body(j, carry):
            pltpu.sync_copy(ni_hbm.at[wid].at[j], nb_v)
            pltpu.sync_copy(hi_hbm.at[wid].at[j], hb_v)
            pltpu.sync_copy(ones_v, dv_sh.at[nb_v], add=True)
            pltpu.sync_copy(ones_v, de_sh.at[hb_v], add=True)
            return carry

        lax.fori_loop(0, K, body, 0)
        plsc.subcore_barrier()

        for off, sz in dv_chunks:
            base = sid * dv_stripe + off
            pltpu.sync_copy(dv_sh.at[pl.ds(base, sz)], ones_v.at[pl.ds(0, sz)])
            pltpu.sync_copy(ones_v.at[pl.ds(0, sz)],
                            dv_out.at[cid].at[pl.ds(base, sz)])
        for off, sz in de_chunks:
            base = sid * de_stripe + off
            pltpu.sync_copy(de_sh.at[pl.ds(base, sz)], ones_v.at[pl.ds(0, sz)])
            pltpu.sync_copy(ones_v.at[pl.ds(0, sz)],
                            de_out.at[cid].at[pl.ds(base, sz)])

    return deg_kernel


def _scale_from_parts(parts, power):
    d = parts[0] + parts[1]
    if power == -1.0:
        s = 1.0 / d
    else:
        s = lax.rsqrt(d)
    s = jnp.where(d > 0, s, 0.0)
    return s[:, 0:1]


def _mlp1_body(x_ref, w_ref, b_ref, dvp_ref, y0, y1):
    s = _scale_from_parts(dvp_ref[...], -0.5)
    y = (jnp.dot(x_ref[...], w_ref[...], preferred_element_type=F32)
         + b_ref[...]) * s
    y0[...] = y[:, :H]
    y1[...] = y[:, H:]


def _escale_body(e0p, e1p, dep_ref, e0, e1):
    s = _scale_from_parts(dep_ref[...], -1.0)
    e0[...] = (e0p[0] + e0p[1]) * s
    e1[...] = (e1p[0] + e1p[1]) * s


def _mlp2_body(z0p, z1p, dvp_ref, w2_ref, b_ref, y0, y1):
    s = _scale_from_parts(dvp_ref[...], -0.5)
    h0 = jnp.maximum((z0p[0] + z0p[1]) * s, 0.0)
    h1 = jnp.maximum((z1p[0] + z1p[1]) * s, 0.0)
    y = (jnp.dot(h0, w2_ref[:H, :], preferred_element_type=F32)
         + jnp.dot(h1, w2_ref[H:, :], preferred_element_type=F32)
         + b_ref[...]) * s
    y0[...] = y[:, :H]
    y1[...] = y[:, H:]


def _final_body(z0p, z1p, dvp_ref, o0, o1):
    s = _scale_from_parts(dvp_ref[...], -0.5)
    o0[...] = (z0p[0] + z0p[1]) * s
    o1[...] = (z1p[0] + z1p[1]) * s


def kernel(X, node_idx, hedge_idx, W1, b1, W2, b2):
    N, D = X.shape
    HID = W1.shape[1]
    OUT = W2.shape[1]
    NNZ = node_idx.shape[0]
    NE = 2500

    def rup(x, m):
        return (x + m - 1) // m * m

    NP = rup(N + 1, NS * B)
    NEP = rup(NE + 1, B)
    K = (NNZ + NW * B - 1) // (NW * B)
    tot = NW * K * B

    npad = tot - NNZ
    ni_pad = (N + jnp.arange(npad, dtype=jnp.int32) % (NP - N))
    hi_pad = (NE + jnp.arange(npad, dtype=jnp.int32) % (NEP - NE))
    ni_flat = jnp.concatenate([node_idx.astype(jnp.int32), ni_pad])
    hi_flat = jnp.concatenate([hedge_idx.astype(jnp.int32), hi_pad])
    ni = ni_flat.reshape(NW, K, B)
    hi = hi_flat.reshape(NW, K, B)

    nb = tot // B
    Kn = rup(int(round(nb * 0.5)) // NS, 2)
    Ks = rup((nb - Kn * NS + NS - 1) // NS, 2)
    Kmax = max(Kn, Ks)

    def asym(flat):
        blocks = flat.reshape(nb, B)
        north = blocks[:Kn * NS].reshape(NS, Kn, B)
        south = blocks[Kn * NS:Kn * NS + Ks * NS]
        south = jnp.concatenate(
            [south, jnp.tile(blocks[-1:], (Ks * NS - (nb - Kn * NS), 1))]
        )[: Ks * NS].reshape(NS, Ks, B) if (nb - Kn * NS) < Ks * NS else \
            south.reshape(NS, Ks, B)
        if Ks < Kmax:
            south = jnp.concatenate(
                [south, jnp.tile(south[:, -1:], (1, Kmax - Ks, 1))], axis=1)
        if Kn < Kmax:
            north = jnp.concatenate(
                [north, jnp.tile(north[:, -1:], (1, Kmax - Kn, 1))], axis=1)
        return jnp.concatenate([north, south], axis=0)

    Xp = jnp.pad(X, ((0, NP - N), (0, 0)))
    b1r = b1.reshape(1, HID)
    b2r = b2.reshape(1, OUT)

    ni_a = asym(ni_flat)
    hi_a = asym(hi_flat)

    dv_parts, de_parts = _make_degree_kernel(K, NP, NEP)(ni, hi)

    RB = 1024
    G = NP // RB
    h_out = lambda R: [jax.ShapeDtypeStruct((R, H), F32)] * 2

    mlp1 = pl.pallas_call(
        _mlp1_body,
        grid=(G,),
        in_specs=[
            pl.BlockSpec((RB, D), lambda i: (i, 0)),
            pl.BlockSpec((D, HID), lambda i: (0, 0)),
            pl.BlockSpec((1, HID), lambda i: (0, 0)),
            pl.BlockSpec((NC, RB, H), lambda i: (0, i, 0)),
        ],
        out_specs=[pl.BlockSpec((RB, H), lambda i: (i, 0))] * 2,
        out_shape=h_out(NP),
    )

    escale = pl.pallas_call(
        _escale_body,
        grid=(1,),
        in_specs=[pl.BlockSpec((NC, NEP, H), lambda i: (0, 0, 0))] * 2
        + [pl.BlockSpec((NC, NEP, H), lambda i: (0, 0, 0))],
        out_specs=[pl.BlockSpec((NEP, H), lambda i: (0, 0))] * 2,
        out_shape=h_out(NEP),
    )

    mlp2 = pl.pallas_call(
        _mlp2_body,
        grid=(G,),
        in_specs=[pl.BlockSpec((NC, RB, H), lambda i: (0, i, 0))] * 2
        + [
            pl.BlockSpec((NC, RB, H), lambda i: (0, i, 0)),
            pl.BlockSpec((HID, OUT), lambda i: (0, 0)),
            pl.BlockSpec((1, OUT), lambda i: (0, 0)),
        ],
        out_specs=[pl.BlockSpec((RB, H), lambda i: (i, 0))] * 2,
        out_shape=h_out(NP),
    )

    final = pl.pallas_call(
        _final_body,
        grid=(G,),
        in_specs=[pl.BlockSpec((NC, RB, H), lambda i: (0, i, 0))] * 2
        + [pl.BlockSpec((NC, RB, H), lambda i: (0, i, 0))],
        out_specs=[pl.BlockSpec((RB, H), lambda i: (i, 0))] * 2,
        out_shape=h_out(NP),
    )

    seg_e = _make_segsum_kernel(Kn, Ks, NP, NEP)
    seg_z = _make_segsum_kernel(Kn, Ks, NEP, NP)

    def laplacian(yh):
        eph = seg_e(*yh, ni_a, hi_a)
        eh = escale(*eph, de_parts)
        return seg_z(*eh, hi_a, ni_a)

    yh = mlp1(Xp, W1, b1r, dv_parts)
    zph = laplacian(yh)
    y2h = mlp2(*zph, dv_parts, W2, b2r)
    zph2 = laplacian(y2h)
    o0, o1 = final(*zph2, dv_parts)
    return jnp.concatenate([o0[:N], o1[:N]], axis=1)


# speedup vs baseline: 1.1012x; 1.0489x over previous
import functools

import jax
import jax.numpy as jnp
from jax import lax
from jax.experimental import pallas as pl
from jax.experimental.pallas import tpu as pltpu
from jax.experimental.pallas import tpu_sc as plsc

NC = 2
NS = 16
NW = NC * NS
B = 128
H = 128

F32 = jnp.float32


def _mesh():
    return plsc.VectorSubcoreMesh(core_axis_name="c", subcore_axis_name="s")


def _zero_vmem(ref, rows, cols):
    z = jnp.zeros((16,), F32)
    cchunks = cols // 16

    def body(i, carry):
        ref[i // cchunks, pl.ds((i % cchunks) * 16, 16)] = z
        return carry

    lax.fori_loop(0, rows * cchunks, body, 0)


def _fill_vmem(ref, rows, cols, val):
    v = jnp.full((16,), val, F32)
    cchunks = cols // 16

    def body(i, carry):
        ref[i // cchunks, pl.ds((i % cchunks) * 16, 16)] = v
        return carry

    lax.fori_loop(0, rows * cchunks, body, 0)


def _stripe_chunks(stripe):
    chunks = []
    off = 0
    while off < stripe:
        sz = min(B, stripe - off)
        chunks.append((off, sz))
        off += sz
    return chunks


def _make_segsum_kernel(Kn, Ks, src_rows, dst_rows):
    stripe = dst_rows // NS
    chunks = _stripe_chunks(stripe)
    out_t = jax.ShapeDtypeStruct((NC, dst_rows, H), F32)
    K = max(Kn, Ks)

    assert Kn % 2 == 0 and Ks % 2 == 0

    @functools.partial(
        pl.kernel,
        out_type=(out_t, out_t),
        mesh=_mesh(),
        scratch_types=[
            pltpu.VMEM((K, B), jnp.int32),
            pltpu.VMEM((B,), jnp.int32),
            pltpu.VMEM((B,), jnp.int32),
            pltpu.VMEM((B, H), F32),
            pltpu.VMEM((B, H), F32),
            pltpu.SemaphoreType.DMA,
            pltpu.SemaphoreType.DMA,
            pltpu.SemaphoreType.DMA,
            pltpu.SemaphoreType.DMA,
            pltpu.VMEM_SHARED((dst_rows, H), F32),
        ],
    )
    def segsum_kernel(s0, s1, gi_hbm, si_hbm, o0, o1,
                      gi_v, sb0, sb1, buf0, buf1, sg0, sg1, si0, si1, acc_sh):
        cid = lax.axis_index("c")
        sid = lax.axis_index("s")
        wid = cid * NS + sid
        kw = jnp.where(cid == 0, Kn, Ks)

        pltpu.sync_copy(gi_hbm.at[wid], gi_v)

        for src_hbm, out_hbm in ((s0, o0), (s1, o1)):
            _zero_vmem(buf0, B, H)
            for off, sz in chunks:
                pltpu.sync_copy(buf0.at[pl.ds(0, sz)],
                                acc_sh.at[pl.ds(sid * stripe + off, sz)])
            plsc.subcore_barrier()

            def start(j, sb, buf, sg, si_sem):
                gdesc = pltpu.async_copy(src_hbm.at[gi_v.at[j]], buf, sg)
                idesc = pltpu.async_copy(si_hbm.at[wid].at[j], sb, si_sem)
                return gdesc, idesc

            def finish(sb, buf, gdesc, idesc):
                gdesc.wait()
                idesc.wait()
                pltpu.sync_copy(buf, acc_sh.at[sb], add=True)

            g0, i0 = start(0, sb0, buf0, sg0, si0)

            def body(i, carry):
                j0 = 2 * i
                g0d, i0d = pltpu.make_async_copy(src_hbm.at[gi_v.at[j0]], buf0, sg0), \
                    pltpu.make_async_copy(si_hbm.at[wid].at[j0], sb0, si0)
                g1d = pltpu.async_copy(src_hbm.at[gi_v.at[j0 + 1]], buf1, sg1)
                i1d = pltpu.async_copy(si_hbm.at[wid].at[j0 + 1], sb1, si1)
                finish(sb0, buf0, g0d, i0d)

                @pl.when(j0 + 2 < kw)
                def _():
                    pltpu.async_copy(src_hbm.at[gi_v.at[j0 + 2]], buf0, sg0)
                    pltpu.async_copy(si_hbm.at[wid].at[j0 + 2], sb0, si0)

                finish(sb1, buf1, g1d, i1d)
                return carry

            lax.fori_loop(0, kw // 2, body, 0)
            plsc.subcore_barrier()

            for off, sz in chunks:
                base = sid * stripe + off
                pltpu.sync_copy(acc_sh.at[pl.ds(base, sz)],
                                buf0.at[pl.ds(0, sz)])
                pltpu.sync_copy(buf0.at[pl.ds(0, sz)],
                                out_hbm.at[cid].at[pl.ds(base, sz)])

    return segsum_kernel


def _make_degree_kernel(K, NP, NEP):
    dv_stripe = NP // NS
    de_stripe = NEP // NS
    dv_chunks = _stripe_chunks(dv_stripe)
    de_chunks = _stripe_chunks(de_stripe)

    @functools.partial(
        pl.kernel,
        out_type=(
            jax.ShapeDtypeStruct((NC, NP, H), F32),
            jax.ShapeDtypeStruct((NC, NEP, H), F32),
        ),
        mesh=_mesh(),
        scratch_types=[
            pltpu.VMEM((B,), jnp.int32),
            pltpu.VMEM((B,), jnp.int32),
            pltpu.VMEM((B,), jnp.int32),
            pltpu.VMEM((B,), jnp.int32),
            pltpu.VMEM((B, H), F32),
            pltpu.SemaphoreType.DMA,
            pltpu.SemaphoreType.DMA,
            pltpu.SemaphoreType.DMA,
            pltpu.SemaphoreType.DMA,
            pltpu.VMEM_SHARED((NP, H), F32),
            pltpu.VMEM_SHARED((NEP, H), F32),
        ],
    )
    def deg_kernel(ni_hbm, hi_hbm, dv_out, de_out,
                   nb0, nb1, hb0, hb1, ones_v, sn0, sn1, sh0, sh1,
                   dv_sh, de_sh):
        cid = lax.axis_index("c")
        sid = lax.axis_index("s")
        wid = cid * NS + sid

        _zero_vmem(ones_v, B, H)
        for off, sz in dv_chunks:
            pltpu.sync_copy(ones_v.at[pl.ds(0, sz)],
                            dv_sh.at[pl.ds(sid * dv_stripe + off, sz)])
        for off, sz in de_chunks:
            pltpu.sync_copy(ones_v.at[pl.ds(0, sz)],
                            de_sh.at[pl.ds(sid * de_stripe + off, sz)])
        _fill_vmem(ones_v, B, H, 1.0)
        plsc.subcore_barrier()

        pltpu.async_copy(ni_hbm.at[wid].at[0], nb0, sn0)
        pltpu.async_copy(hi_hbm.at[wid].at[0], hb0, sh0)

        def body(i, carry):
            j0 = 2 * i
            n0d = pltpu.make_async_copy(ni_hbm.at[wid].at[j0], nb0, sn0)
            h0d = pltpu.make_async_copy(hi_hbm.at[wid].at[j0], hb0, sh0)
            n1d = pltpu.async_copy(ni_hbm.at[wid].at[j0 + 1], nb1, sn1)
            h1d = pltpu.async_copy(hi_hbm.at[wid].at[j0 + 1], hb1, sh1)
            n0d.wait()
            h0d.wait()
            pltpu.sync_copy(ones_v, dv_sh.at[nb0], add=True)
            pltpu.sync_copy(ones_v, de_sh.at[hb0], add=True)

            @pl.when(j0 + 2 < K)
            def _():
                pltpu.async_copy(ni_hbm.at[wid].at[j0 + 2], nb0, sn0)
                pltpu.async_copy(hi_hbm.at[wid].at[j0 + 2], hb0, sh0)

            n1d.wait()
            h1d.wait()
            pltpu.sync_copy(ones_v, dv_sh.at[nb1], add=True)
            pltpu.sync_copy(ones_v, de_sh.at[hb1], add=True)
            return carry

        assert K % 2 == 0
        lax.fori_loop(0, K // 2, body, 0)
        plsc.subcore_barrier()

        for off, sz in dv_chunks:
            base = sid * dv_stripe + off
            pltpu.sync_copy(dv_sh.at[pl.ds(base, sz)], ones_v.at[pl.ds(0, sz)])
            pltpu.sync_copy(ones_v.at[pl.ds(0, sz)],
                            dv_out.at[cid].at[pl.ds(base, sz)])
        for off, sz in de_chunks:
            base = sid * de_stripe + off
            pltpu.sync_copy(de_sh.at[pl.ds(base, sz)], ones_v.at[pl.ds(0, sz)])
            pltpu.sync_copy(ones_v.at[pl.ds(0, sz)],
                            de_out.at[cid].at[pl.ds(base, sz)])

    return deg_kernel


def _scale_from_parts(parts, power):
    d = parts[0] + parts[1]
    if power == -1.0:
        s = 1.0 / d
    else:
        s = lax.rsqrt(d)
    s = jnp.where(d > 0, s, 0.0)
    return s[:, 0:1]


def _mlp1_body(x_ref, w_ref, b_ref, dvp_ref, y0, y1):
    s = _scale_from_parts(dvp_ref[...], -0.5)
    y = (jnp.dot(x_ref[...], w_ref[...], preferred_element_type=F32)
         + b_ref[...]) * s
    y0[...] = y[:, :H]
    y1[...] = y[:, H:]


def _escale_body(e0p, e1p, dep_ref, e0, e1):
    s = _scale_from_parts(dep_ref[...], -1.0)
    e0[...] = (e0p[0] + e0p[1]) * s
    e1[...] = (e1p[0] + e1p[1]) * s


def _mlp2_body(z0p, z1p, dvp_ref, w2_ref, b_ref, y0, y1):
    s = _scale_from_parts(dvp_ref[...], -0.5)
    h0 = jnp.maximum((z0p[0] + z0p[1]) * s, 0.0)
    h1 = jnp.maximum((z1p[0] + z1p[1]) * s, 0.0)
    y = (jnp.dot(h0, w2_ref[:H, :], preferred_element_type=F32)
         + jnp.dot(h1, w2_ref[H:, :], preferred_element_type=F32)
         + b_ref[...]) * s
    y0[...] = y[:, :H]
    y1[...] = y[:, H:]


def _final_body(z0p, z1p, dvp_ref, o0, o1):
    s = _scale_from_parts(dvp_ref[...], -0.5)
    o0[...] = (z0p[0] + z0p[1]) * s
    o1[...] = (z1p[0] + z1p[1]) * s


def kernel(X, node_idx, hedge_idx, W1, b1, W2, b2):
    N, D = X.shape
    HID = W1.shape[1]
    OUT = W2.shape[1]
    NNZ = node_idx.shape[0]
    NE = 2500

    def rup(x, m):
        return (x + m - 1) // m * m

    NP = rup(N + 1, NS * B)
    NEP = rup(NE + 1, B)
    K = (NNZ + NW * B - 1) // (NW * B)
    tot = NW * K * B

    npad = tot - NNZ
    ni_pad = (N + jnp.arange(npad, dtype=jnp.int32) % (NP - N))
    hi_pad = (NE + jnp.arange(npad, dtype=jnp.int32) % (NEP - NE))
    ni_flat = jnp.concatenate([node_idx.astype(jnp.int32), ni_pad])
    hi_flat = jnp.concatenate([hedge_idx.astype(jnp.int32), hi_pad])
    ni = ni_flat.reshape(NW, K, B)
    hi = hi_flat.reshape(NW, K, B)

    nb = tot // B
    Kn = rup(int(round(nb * 0.5)) // NS, 2)
    Ks = rup((nb - Kn * NS + NS - 1) // NS, 2)
    Kmax = max(Kn, Ks)

    def asym(flat):
        blocks = flat.reshape(nb, B)
        north = blocks[:Kn * NS].reshape(NS, Kn, B)
        south = blocks[Kn * NS:Kn * NS + Ks * NS]
        south = jnp.concatenate(
            [south, jnp.tile(blocks[-1:], (Ks * NS - (nb - Kn * NS), 1))]
        )[: Ks * NS].reshape(NS, Ks, B) if (nb - Kn * NS) < Ks * NS else \
            south.reshape(NS, Ks, B)
        if Ks < Kmax:
            south = jnp.concatenate(
                [south, jnp.tile(south[:, -1:], (1, Kmax - Ks, 1))], axis=1)
        if Kn < Kmax:
            north = jnp.concatenate(
                [north, jnp.tile(north[:, -1:], (1, Kmax - Kn, 1))], axis=1)
        return jnp.concatenate([north, south], axis=0)

    Xp = jnp.pad(X, ((0, NP - N), (0, 0)))
    b1r = b1.reshape(1, HID)
    b2r = b2.reshape(1, OUT)

    ni_a = asym(ni_flat)
    hi_a = asym(hi_flat)

    dv_parts, de_parts = _make_degree_kernel(K, NP, NEP)(ni, hi)

    RB = 1024
    G = NP // RB
    h_out = lambda R: [jax.ShapeDtypeStruct((R, H), F32)] * 2

    mlp1 = pl.pallas_call(
        _mlp1_body,
        grid=(G,),
        in_specs=[
            pl.BlockSpec((RB, D), lambda i: (i, 0)),
            pl.BlockSpec((D, HID), lambda i: (0, 0)),
            pl.BlockSpec((1, HID), lambda i: (0, 0)),
            pl.BlockSpec((NC, RB, H), lambda i: (0, i, 0)),
        ],
        out_specs=[pl.BlockSpec((RB, H), lambda i: (i, 0))] * 2,
        out_shape=h_out(NP),
    )

    escale = pl.pallas_call(
        _escale_body,
        grid=(1,),
        in_specs=[pl.BlockSpec((NC, NEP, H), lambda i: (0, 0, 0))] * 2
        + [pl.BlockSpec((NC, NEP, H), lambda i: (0, 0, 0))],
        out_specs=[pl.BlockSpec((NEP, H), lambda i: (0, 0))] * 2,
        out_shape=h_out(NEP),
    )

    mlp2 = pl.pallas_call(
        _mlp2_body,
        grid=(G,),
        in_specs=[pl.BlockSpec((NC, RB, H), lambda i: (0, i, 0))] * 2
        + [
            pl.BlockSpec((NC, RB, H), lambda i: (0, i, 0)),
            pl.BlockSpec((HID, OUT), lambda i: (0, 0)),
            pl.BlockSpec((1, OUT), lambda i: (0, 0)),
        ],
        out_specs=[pl.BlockSpec((RB, H), lambda i: (i, 0))] * 2,
        out_shape=h_out(NP),
    )

    final = pl.pallas_call(
        _final_body,
        grid=(G,),
        in_specs=[pl.BlockSpec((NC, RB, H), lambda i: (0, i, 0))] * 2
        + [pl.BlockSpec((NC, RB, H), lambda i: (0, i, 0))],
        out_specs=[pl.BlockSpec((RB, H), lambda i: (i, 0))] * 2,
        out_shape=h_out(NP),
    )

    seg_e = _make_segsum_kernel(Kn, Ks, NP, NEP)
    seg_z = _make_segsum_kernel(Kn, Ks, NEP, NP)

    def laplacian(yh):
        eph = seg_e(*yh, ni_a, hi_a)
        eh = escale(*eph, de_parts)
        return seg_z(*eh, hi_a, ni_a)

    yh = mlp1(Xp, W1, b1r, dv_parts)
    zph = laplacian(yh)
    y2h = mlp2(*zph, dv_parts, W2, b2r)
    zph2 = laplacian(y2h)
    o0, o1 = final(*zph2, dv_parts)
    return jnp.concatenate([o0[:N], o1[:N]], axis=1)
